# baseline (speedup 1.0000x reference)
import functools

import jax
import jax.numpy as jnp
from jax import lax
from jax.experimental import pallas as pl
from jax.experimental.pallas import tpu as pltpu
from jax.experimental.pallas import tpu_sc as plsc

D_MODEL = 64
SCALE = 8.0
BBLK = 128
NUM_WORKERS = 32
NBUF = 4


def kernel(x, table):
    n_seq, seq_len = x.shape
    n_bblk = n_seq // BBLK
    n_tt = seq_len // 8

    xl = x.T.reshape(n_tt, 8, n_bblk, BBLK).transpose(0, 2, 1, 3)

    mesh = plsc.VectorSubcoreMesh(core_axis_name="c", subcore_axis_name="s")

    @functools.partial(
        pl.kernel,
        mesh=mesh,
        compiler_params=pltpu.CompilerParams(
            use_tc_tiling_on_sc=False, needs_layout_passes=False
        ),
        out_type=jax.ShapeDtypeStruct(
            (seq_len, D_MODEL // 8, n_bblk, 8, BBLK), jnp.float32
        ),
        scratch_types=[
            pltpu.VMEM((n_tt, 8, BBLK), jnp.int32),
            pltpu.VMEM((NBUF, BBLK, D_MODEL), jnp.float32),
            pltpu.VMEM((2, D_MODEL // 8, 8, BBLK), jnp.float32),
            [pltpu.SemaphoreType.DMA] * NBUF,
            [pltpu.SemaphoreType.DMA] * 2,
        ],
    )
    def emb_kernel(x_hbm, table_hbm, out_hbm, idx_v, rows_v, tp_v, gsem, ssem):
        wid = lax.axis_index("s") * 2 + lax.axis_index("c")
        pltpu.sync_copy(x_hbm.at[:, wid, :, :], idx_v)

        def idx_slice(t):
            return idx_v.at[t // 8, t % 8]

        for k in range(NBUF - 1):
            pltpu.make_async_copy(
                table_hbm.at[idx_slice(k)], rows_v.at[k], gsem[k]
            ).start()

        lanes = lax.iota(jnp.int32, 16)

        def outer(it, carry):
            for k in range(NBUF):
                t = it * NBUF + k
                kt = k % 2
                buf = rows_v.at[k]
                tbuf = tp_v.at[kt]
                pltpu.make_async_copy(
                    table_hbm.at[idx_slice(0)], buf, gsem[k]
                ).wait()

                def drain_tbuf():
                    pltpu.make_async_copy(
                        tp_v.at[kt], out_hbm.at[0, :, 0], ssem[kt]
                    ).wait()

                if k >= 2:
                    drain_tbuf()
                else:
                    pl.when(it >= 1)(drain_tbuf)

                @plsc.parallel_loop(0, BBLK, unroll=4)
                def tp_tok(b):
                    bb = jnp.full((16,), b, jnp.int32)
                    for j16 in range(D_MODEL // 16):
                        jids = j16 * 16 + lanes
                        v = buf[b, pl.ds(j16 * 16, 16)]
                        plsc.store_scatter(
                            tbuf, [jids >> 3, jids & 7, bb], v * SCALE
                        )

                pltpu.make_async_copy(
                    tbuf, out_hbm.at[t, :, wid], ssem[kt]
                ).start()

                gn = t + NBUF - 1

                @pl.when(gn < seq_len)
                def _refill():
                    pltpu.make_async_copy(
                        table_hbm.at[idx_slice(gn)],
                        rows_v.at[(k + NBUF - 1) % NBUF],
                        gsem[(k + NBUF - 1) % NBUF],
                    ).start()

            return carry

        lax.fori_loop(0, seq_len // NBUF, outer, 0)

        for kt in range(2):
            pltpu.make_async_copy(
                tp_v.at[kt], out_hbm.at[0, :, 0], ssem[kt]
            ).wait()

    out5 = emb_kernel(xl, table)
    return (
        out5.transpose(2, 4, 0, 1, 3).reshape(n_seq, seq_len, D_MODEL)
    )


# speedup vs baseline: 1.7326x; 1.7326x over previous
import functools

import jax
import jax.numpy as jnp
from jax import lax
from jax.experimental import pallas as pl
from jax.experimental.pallas import tpu as pltpu
from jax.experimental.pallas import tpu_sc as plsc

D_MODEL = 64
SCALE = 8.0
BBLK = 128
NUM_WORKERS = 32
NBUF = 4


def kernel(x, table):
    n_seq, seq_len = x.shape
    n_bblk = n_seq // BBLK
    n_tt = seq_len // 8

    xl = x.T.reshape(n_tt, 8, n_bblk, BBLK).transpose(0, 2, 1, 3)

    mesh = plsc.VectorSubcoreMesh(core_axis_name="c", subcore_axis_name="s")

    @functools.partial(
        pl.kernel,
        mesh=mesh,
        compiler_params=pltpu.CompilerParams(
            use_tc_tiling_on_sc=False, needs_layout_passes=False
        ),
        out_type=jax.ShapeDtypeStruct(
            (seq_len, D_MODEL // 8, n_bblk, 8, BBLK), jnp.float32
        ),
        scratch_types=[
            pltpu.VMEM((n_tt, 8, BBLK), jnp.int32),
            pltpu.VMEM((NBUF, BBLK, D_MODEL), jnp.float32),
            pltpu.VMEM((2, D_MODEL // 8, 8, BBLK + 1), jnp.float32),
            [pltpu.SemaphoreType.DMA] * NBUF,
            [pltpu.SemaphoreType.DMA] * 2,
        ],
    )
    def emb_kernel(x_hbm, table_hbm, out_hbm, idx_v, rows_v, tp_v, gsem, ssem):
        wid = lax.axis_index("s") * 2 + lax.axis_index("c")
        pltpu.sync_copy(x_hbm.at[:, wid, :, :], idx_v)

        def idx_slice(t):
            return idx_v.at[t // 8, t % 8]

        for k in range(NBUF - 1):
            pltpu.make_async_copy(
                table_hbm.at[idx_slice(k)], rows_v.at[k], gsem[k]
            ).start()

        lanes = lax.iota(jnp.int32, 16)

        def outer(it, carry):
            for k in range(NBUF):
                t = it * NBUF + k
                kt = k % 2
                buf = rows_v.at[k]
                tbuf = tp_v.at[kt]
                pltpu.make_async_copy(
                    table_hbm.at[idx_slice(0)], buf, gsem[k]
                ).wait()

                def drain_tbuf():
                    pltpu.make_async_copy(
                        tp_v.at[kt].at[:, :, pl.ds(0, BBLK)],
                        out_hbm.at[0, :, 0],
                        ssem[kt],
                    ).wait()

                if k >= 2:
                    drain_tbuf()
                else:
                    pl.when(it >= 1)(drain_tbuf)

                @plsc.parallel_loop(0, BBLK, unroll=4)
                def tp_tok(b):
                    bb = jnp.full((16,), b, jnp.int32)
                    for j16 in range(D_MODEL // 16):
                        jids = j16 * 16 + lanes
                        v = buf[b, pl.ds(j16 * 16, 16)]
                        plsc.store_scatter(
                            tbuf, [jids >> 3, jids & 7, bb], v * SCALE
                        )

                pltpu.make_async_copy(
                    tbuf.at[:, :, pl.ds(0, BBLK)],
                    out_hbm.at[t, :, wid],
                    ssem[kt],
                ).start()

                gn = t + NBUF - 1

                @pl.when(gn < seq_len)
                def _refill():
                    pltpu.make_async_copy(
                        table_hbm.at[idx_slice(gn)],
                        rows_v.at[(k + NBUF - 1) % NBUF],
                        gsem[(k + NBUF - 1) % NBUF],
                    ).start()

            return carry

        lax.fori_loop(0, seq_len // NBUF, outer, 0)

        for kt in range(2):
            pltpu.make_async_copy(
                tp_v.at[kt].at[:, :, pl.ds(0, BBLK)],
                out_hbm.at[0, :, 0],
                ssem[kt],
            ).wait()

    out5 = emb_kernel(xl, table)
    return (
        out5.transpose(2, 4, 0, 1, 3).reshape(n_seq, seq_len, D_MODEL)
    )
